# baseline (speedup 1.0000x reference)
import functools

import jax
import jax.numpy as jnp
from jax.experimental import pallas as pl
from jax.experimental.pallas import tpu as pltpu

_D = 64


_WAYS = 2


def _chunk(e):
    c = min(e, 2000)
    while e % (_WAYS * c):
        c -= 1
    return c


_PARAMS = pltpu.CompilerParams(
    dimension_semantics=("parallel", "arbitrary"))


def _degree_call(edges, n_u, n_s):
    e = edges.shape[1]
    c = _chunk(e)
    nb = e // c

    def body(idx_ref, ud_ref, sd_ref):
        pid = pl.program_id(0)

        @pl.when(pid == 0)
        def _():
            ud_ref[...] = jnp.zeros_like(ud_ref)
            sd_ref[...] = jnp.zeros_like(sd_ref)

        ones = jnp.ones((1, _D), jnp.float32)

        def step(i, carry):
            u = idx_ref[0, 0, i]
            s = idx_ref[0, 1, i]
            ud_ref[pl.ds(u, 1), :] = ud_ref[pl.ds(u, 1), :] + ones
            sd_ref[pl.ds(s, 1), :] = sd_ref[pl.ds(s, 1), :] + ones
            return carry

        jax.lax.fori_loop(0, c, step, 0)

        @pl.when(pid == nb - 1)
        def _():
            ud = ud_ref[...]
            ud_ref[...] = jnp.where(ud > 0, jax.lax.rsqrt(ud), 0.0)
            sd = sd_ref[...]
            sd_ref[...] = jnp.where(sd > 0, jax.lax.rsqrt(sd), 0.0)

    return pl.pallas_call(
        body,
        grid=(nb,),
        in_specs=[pl.BlockSpec((1, 2, c), lambda i: (i, 0, 0),
                               memory_space=pltpu.SMEM)],
        out_specs=[pl.BlockSpec((n_u, _D), lambda i: (0, 0)),
                   pl.BlockSpec((n_s, _D), lambda i: (0, 0))],
        out_shape=[jax.ShapeDtypeStruct((n_u, _D), jnp.float32),
                   jax.ShapeDtypeStruct((n_s, _D), jnp.float32)],
    )(edges.reshape(2, nb, c).swapaxes(0, 1))


def _conv_call(x, edges, w):
    m = x.shape[0]
    e = edges.shape[1]
    c = _chunk(e)
    nb = e // c

    def body(idx_ref, w_ref, x_ref, out_ref):
        @pl.when(pl.program_id(0) == 0)
        def _():
            out_ref[...] = jnp.zeros_like(out_ref)

        def step(i, carry):
            d = idx_ref[0, 0, i]
            s = idx_ref[0, 1, i]
            wv = w_ref[0, 0, i]
            out_ref[pl.ds(d, 1), :] = (out_ref[pl.ds(d, 1), :]
                                       + x_ref[pl.ds(s, 1), :] * wv)
            return carry

        jax.lax.fori_loop(0, c, step, 0)

    return pl.pallas_call(
        body,
        grid=(nb,),
        in_specs=[pl.BlockSpec((1, 2, c), lambda i: (i, 0, 0),
                               memory_space=pltpu.SMEM),
                  pl.BlockSpec((1, 1, c), lambda i: (i, 0, 0),
                               memory_space=pltpu.SMEM),
                  pl.BlockSpec((m, _D), lambda i: (0, 0))],
        out_specs=pl.BlockSpec((m, _D), lambda i: (0, 0)),
        out_shape=jax.ShapeDtypeStruct((m, _D), jnp.float32),
    )(edges.reshape(2, nb, c).swapaxes(0, 1), w.reshape(nb, 1, c), x)


def _scatter_call(edges, src_scaled, dst_scale):
    n_out = dst_scale.shape[0]
    m_src = src_scaled.shape[0]
    e = edges.shape[1]
    c = _chunk(e)
    nb = e // c
    half = nb // _WAYS

    def body(idx_ref, src_ref, out_ref):
        @pl.when(pl.program_id(1) == 0)
        def _():
            out_ref[...] = jnp.zeros_like(out_ref)

        def step(i, carry):
            d = idx_ref[0, 0, i]
            s = idx_ref[0, 1, i]
            out_ref[0, pl.ds(d, 1), :] = (out_ref[0, pl.ds(d, 1), :]
                                          + src_ref[pl.ds(s, 1), :])
            return carry

        jax.lax.fori_loop(0, c, step, 0)

    parts = pl.pallas_call(
        body,
        grid=(_WAYS, half),
        in_specs=[pl.BlockSpec((1, 2, c), lambda i, j: (i * half + j, 0, 0),
                               memory_space=pltpu.SMEM),
                  pl.BlockSpec((m_src, _D), lambda i, j: (0, 0))],
        out_specs=pl.BlockSpec((1, n_out, _D), lambda i, j: (i, 0, 0)),
        out_shape=jax.ShapeDtypeStruct((_WAYS, n_out, _D), jnp.float32),
        compiler_params=_PARAMS,
    )(edges.reshape(2, nb, c).swapaxes(0, 1), src_scaled)
    return _comb_mul_call(parts, dst_scale)


def _fuse_call(x, c1, c2, c3, w1, w2):
    m = x.shape[0]
    blk = 512
    nb = (m + blk - 1) // blk
    w1a = w1[:_D, 0].reshape(1, _D)
    w1b = w1[_D:, 0].reshape(1, _D)
    w2r = w2[:, 0].reshape(1, _D)

    def body(x_ref, c1_ref, c2_ref, c3_ref, a_ref, b_ref, v_ref, out_ref):
        x_b = x_ref[...]
        wa = a_ref[...]
        wb = b_ref[...]
        wv = v_ref[...]

        def score(c_b):
            t1 = jnp.sum(x_b * wa + c_b * wb, axis=1, keepdims=True)
            t2 = jnp.sum(x_b * c_b * wv, axis=1, keepdims=True)
            return jnp.exp(jnp.tanh(t1) + jnp.tanh(t2))

        cb1 = c1_ref[...]
        cb2 = c2_ref[...]
        cb3 = c3_ref[...]
        s1 = score(cb1)
        s2 = score(cb2)
        s3 = score(cb3)
        tot = s1 + s2 + s3
        out_ref[...] = x_b + (cb1 * s1 + cb2 * s2 + cb3 * s3) / tot

    row_spec = pl.BlockSpec((blk, _D), lambda i: (i, 0))
    w_spec = pl.BlockSpec((1, _D), lambda i: (0, 0))
    return pl.pallas_call(
        body,
        grid=(nb,),
        in_specs=[row_spec, row_spec, row_spec, row_spec,
                  w_spec, w_spec, w_spec],
        out_specs=row_spec,
        out_shape=jax.ShapeDtypeStruct((m, _D), jnp.float32),
    )(x, c1, c2, c3, w1a, w1b, w2r)


def _comb_mul_call(parts, sc):
    n = sc.shape[0]
    blk = 512
    nb = (n + blk - 1) // blk

    def body(a_ref, b_ref, s_ref, out_ref):
        out_ref[...] = (a_ref[...] + b_ref[...]) * s_ref[...]

    spec = pl.BlockSpec((blk, _D), lambda i: (i, 0))
    return pl.pallas_call(
        body, grid=(nb,), in_specs=[spec, spec, spec], out_specs=spec,
        out_shape=jax.ShapeDtypeStruct((n, _D), jnp.float32),
    )(parts[0], parts[1], sc)


def _mul_call(a, b):
    m = a.shape[0]
    blk = 512
    nb = (m + blk - 1) // blk

    def body(a_ref, b_ref, out_ref):
        out_ref[...] = a_ref[...] * b_ref[...]

    spec = pl.BlockSpec((blk, _D), lambda i: (i, 0))
    return pl.pallas_call(
        body, grid=(nb,), in_specs=[spec, spec], out_specs=spec,
        out_shape=jax.ShapeDtypeStruct((m, _D), jnp.float32))(a, b)


def _addscale_call(a, b, scale):
    m = a.shape[0]
    blk = 512
    nb = (m + blk - 1) // blk

    def body(a_ref, b_ref, out_ref):
        out_ref[...] = (a_ref[...] + b_ref[...]) * scale

    spec = pl.BlockSpec((blk, _D), lambda i: (i, 0))
    return pl.pallas_call(
        body, grid=(nb,), in_specs=[spec, spec], out_specs=spec,
        out_shape=jax.ShapeDtypeStruct((m, _D), jnp.float32))(a, b)


@jax.jit
def kernel(user_spot, city_edge_index, city_edge_weight, station_edge_index,
           station_edge_weight, category_edge_index, category_edge_weight,
           user_emb, spot_emb, W1_pre, W2_pre):
    n_users = user_emb.shape[0]
    m_spots = spot_emb.shape[0]
    num_layers = 3

    c_cat = _conv_call(spot_emb, category_edge_index, category_edge_weight)
    c_cit = _conv_call(spot_emb, city_edge_index, city_edge_weight)
    c_sta = _conv_call(spot_emb, station_edge_index, station_edge_weight)
    spot_x = _fuse_call(spot_emb, c_cat, c_cit, c_sta, W1_pre, W2_pre)
    user_x = user_emb

    usd, ssd = _degree_call(user_spot, n_users, m_spots)

    us_u = user_spot
    us_s = jnp.stack([user_spot[1], user_spot[0]])

    spot_tot = spot_x
    user_tot = user_x
    for _ in range(num_layers):
        spot_s = _mul_call(spot_x, ssd)
        user_s = _mul_call(user_x, usd)
        user_x = _scatter_call(us_u, spot_s, usd)
        spot_x = _scatter_call(us_s, user_s, ssd)
        user_tot = _addscale_call(user_tot, user_x, 1.0)
        spot_tot = _addscale_call(spot_tot, spot_x, 1.0)

    inv = 1.0 / (num_layers + 1)
    spot_out = _addscale_call(spot_tot, jnp.zeros_like(spot_tot), inv)
    user_out = _addscale_call(user_tot, jnp.zeros_like(user_tot), inv)
    return spot_out, user_out


# speedup vs baseline: 1.4528x; 1.4528x over previous
import functools

import jax
import jax.numpy as jnp
from jax.experimental import pallas as pl
from jax.experimental.pallas import tpu as pltpu

_D = 64


def _chunk(e):
    c = min(e, 2000)
    while e % c:
        c -= 1
    return c


def _degree_call(edges, n_u, n_s):
    e = edges.shape[1]
    c = _chunk(e)
    nb = e // c

    def body(idx_ref, ud_ref, sd_ref):
        pid = pl.program_id(0)

        @pl.when(pid == 0)
        def _():
            ud_ref[...] = jnp.zeros_like(ud_ref)
            sd_ref[...] = jnp.zeros_like(sd_ref)

        ones = jnp.ones((1, _D), jnp.float32)

        def step(i, carry):
            u = idx_ref[0, 0, i]
            s = idx_ref[0, 1, i]
            ud_ref[pl.ds(u, 1), :] = ud_ref[pl.ds(u, 1), :] + ones
            sd_ref[pl.ds(s, 1), :] = sd_ref[pl.ds(s, 1), :] + ones
            return carry

        jax.lax.fori_loop(0, c, step, 0)

        @pl.when(pid == nb - 1)
        def _():
            ud = ud_ref[...]
            ud_ref[...] = jnp.where(ud > 0, jax.lax.rsqrt(ud), 0.0)
            sd = sd_ref[...]
            sd_ref[...] = jnp.where(sd > 0, jax.lax.rsqrt(sd), 0.0)

    return pl.pallas_call(
        body,
        grid=(nb,),
        in_specs=[pl.BlockSpec((1, 2, c), lambda i: (i, 0, 0),
                               memory_space=pltpu.SMEM)],
        out_specs=[pl.BlockSpec((n_u, _D), lambda i: (0, 0)),
                   pl.BlockSpec((n_s, _D), lambda i: (0, 0))],
        out_shape=[jax.ShapeDtypeStruct((n_u, _D), jnp.float32),
                   jax.ShapeDtypeStruct((n_s, _D), jnp.float32)],
    )(edges.reshape(2, nb, c).swapaxes(0, 1))


def _conv_call(x, edges, w):
    m = x.shape[0]
    e = edges.shape[1]
    c = _chunk(e)
    nb = e // c

    def body(idx_ref, w_ref, x_ref, out_ref):
        @pl.when(pl.program_id(0) == 0)
        def _():
            out_ref[...] = jnp.zeros_like(out_ref)

        def step(i, carry):
            d = idx_ref[0, 0, i]
            s = idx_ref[0, 1, i]
            wv = w_ref[0, 0, i]
            out_ref[pl.ds(d, 1), :] = (out_ref[pl.ds(d, 1), :]
                                       + x_ref[pl.ds(s, 1), :] * wv)
            return carry

        jax.lax.fori_loop(0, c, step, 0)

    return pl.pallas_call(
        body,
        grid=(nb,),
        in_specs=[pl.BlockSpec((1, 2, c), lambda i: (i, 0, 0),
                               memory_space=pltpu.SMEM),
                  pl.BlockSpec((1, 1, c), lambda i: (i, 0, 0),
                               memory_space=pltpu.SMEM),
                  pl.BlockSpec((m, _D), lambda i: (0, 0))],
        out_specs=pl.BlockSpec((m, _D), lambda i: (0, 0)),
        out_shape=jax.ShapeDtypeStruct((m, _D), jnp.float32),
    )(edges.reshape(2, nb, c).swapaxes(0, 1), w.reshape(nb, 1, c), x)


def _scatter_call(edges, src_scaled, dst_scale):
    n_out = dst_scale.shape[0]
    m_src = src_scaled.shape[0]
    e = edges.shape[1]
    c = _chunk(e)
    nb = e // c
    unroll = 4
    while c % unroll:
        unroll -= 1

    def body(idx_ref, src_ref, sc_ref, out_ref):
        pid = pl.program_id(0)

        @pl.when(pid == 0)
        def _():
            out_ref[...] = jnp.zeros_like(out_ref)

        def step(i, carry):
            base = i * unroll
            for k in range(unroll):
                d = idx_ref[0, 0, base + k]
                s = idx_ref[0, 1, base + k]
                out_ref[pl.ds(d, 1), :] = (out_ref[pl.ds(d, 1), :]
                                           + src_ref[pl.ds(s, 1), :])
            return carry

        jax.lax.fori_loop(0, c // unroll, step, 0)

        @pl.when(pid == nb - 1)
        def _():
            out_ref[...] = out_ref[...] * sc_ref[...]

    return pl.pallas_call(
        body,
        grid=(nb,),
        in_specs=[pl.BlockSpec((1, 2, c), lambda i: (i, 0, 0),
                               memory_space=pltpu.SMEM),
                  pl.BlockSpec((m_src, _D), lambda i: (0, 0)),
                  pl.BlockSpec((n_out, _D), lambda i: (0, 0))],
        out_specs=pl.BlockSpec((n_out, _D), lambda i: (0, 0)),
        out_shape=jax.ShapeDtypeStruct((n_out, _D), jnp.float32),
    )(edges.reshape(2, nb, c).swapaxes(0, 1), src_scaled, dst_scale)


def _fuse_call(x, c1, c2, c3, w1, w2):
    m = x.shape[0]
    blk = 512
    nb = (m + blk - 1) // blk
    w1a = w1[:_D, 0].reshape(1, _D)
    w1b = w1[_D:, 0].reshape(1, _D)
    w2r = w2[:, 0].reshape(1, _D)

    def body(x_ref, c1_ref, c2_ref, c3_ref, a_ref, b_ref, v_ref, out_ref):
        x_b = x_ref[...]
        wa = a_ref[...]
        wb = b_ref[...]
        wv = v_ref[...]

        def score(c_b):
            t1 = jnp.sum(x_b * wa + c_b * wb, axis=1, keepdims=True)
            t2 = jnp.sum(x_b * c_b * wv, axis=1, keepdims=True)
            return jnp.exp(jnp.tanh(t1) + jnp.tanh(t2))

        cb1 = c1_ref[...]
        cb2 = c2_ref[...]
        cb3 = c3_ref[...]
        s1 = score(cb1)
        s2 = score(cb2)
        s3 = score(cb3)
        tot = s1 + s2 + s3
        out_ref[...] = x_b + (cb1 * s1 + cb2 * s2 + cb3 * s3) / tot

    row_spec = pl.BlockSpec((blk, _D), lambda i: (i, 0))
    w_spec = pl.BlockSpec((1, _D), lambda i: (0, 0))
    return pl.pallas_call(
        body,
        grid=(nb,),
        in_specs=[row_spec, row_spec, row_spec, row_spec,
                  w_spec, w_spec, w_spec],
        out_specs=row_spec,
        out_shape=jax.ShapeDtypeStruct((m, _D), jnp.float32),
    )(x, c1, c2, c3, w1a, w1b, w2r)


def _mul_call(a, b):
    m = a.shape[0]
    blk = 512
    nb = (m + blk - 1) // blk

    def body(a_ref, b_ref, out_ref):
        out_ref[...] = a_ref[...] * b_ref[...]

    spec = pl.BlockSpec((blk, _D), lambda i: (i, 0))
    return pl.pallas_call(
        body, grid=(nb,), in_specs=[spec, spec], out_specs=spec,
        out_shape=jax.ShapeDtypeStruct((m, _D), jnp.float32))(a, b)


def _addscale_call(a, b, scale):
    m = a.shape[0]
    blk = 512
    nb = (m + blk - 1) // blk

    def body(a_ref, b_ref, out_ref):
        out_ref[...] = (a_ref[...] + b_ref[...]) * scale

    spec = pl.BlockSpec((blk, _D), lambda i: (i, 0))
    return pl.pallas_call(
        body, grid=(nb,), in_specs=[spec, spec], out_specs=spec,
        out_shape=jax.ShapeDtypeStruct((m, _D), jnp.float32))(a, b)


@jax.jit
def kernel(user_spot, city_edge_index, city_edge_weight, station_edge_index,
           station_edge_weight, category_edge_index, category_edge_weight,
           user_emb, spot_emb, W1_pre, W2_pre):
    n_users = user_emb.shape[0]
    m_spots = spot_emb.shape[0]
    num_layers = 3

    c_cat = _conv_call(spot_emb, category_edge_index, category_edge_weight)
    c_cit = _conv_call(spot_emb, city_edge_index, city_edge_weight)
    c_sta = _conv_call(spot_emb, station_edge_index, station_edge_weight)
    spot_x = _fuse_call(spot_emb, c_cat, c_cit, c_sta, W1_pre, W2_pre)
    user_x = user_emb

    usd, ssd = _degree_call(user_spot, n_users, m_spots)

    us_u = user_spot
    us_s = jnp.stack([user_spot[1], user_spot[0]])

    spot_tot = spot_x
    user_tot = user_x
    for _ in range(num_layers):
        spot_s = _mul_call(spot_x, ssd)
        user_s = _mul_call(user_x, usd)
        user_x = _scatter_call(us_u, spot_s, usd)
        spot_x = _scatter_call(us_s, user_s, ssd)
        user_tot = _addscale_call(user_tot, user_x, 1.0)
        spot_tot = _addscale_call(spot_tot, spot_x, 1.0)

    inv = 1.0 / (num_layers + 1)
    spot_out = _addscale_call(spot_tot, jnp.zeros_like(spot_tot), inv)
    user_out = _addscale_call(user_tot, jnp.zeros_like(user_tot), inv)
    return spot_out, user_out


# speedup vs baseline: 1.6937x; 1.1658x over previous
import functools

import jax
import jax.numpy as jnp
from jax.experimental import pallas as pl
from jax.experimental.pallas import tpu as pltpu

_D = 64


def _chunk(e):
    c = min(e, 2000)
    while e % c:
        c -= 1
    return c


def _degree_call(edges, n_u, n_s):
    e = edges.shape[1]
    c = _chunk(e)
    nb = e // c

    def body(idx_ref, ud_ref, sd_ref):
        pid = pl.program_id(0)

        @pl.when(pid == 0)
        def _():
            ud_ref[...] = jnp.zeros_like(ud_ref)
            sd_ref[...] = jnp.zeros_like(sd_ref)

        ones = jnp.ones((1, _D), jnp.float32)
        un = 4
        while c % un:
            un -= 1

        def step(i, carry):
            base = i * un
            for k in range(un):
                u = idx_ref[0, 0, base + k]
                s = idx_ref[0, 1, base + k]
                ud_ref[pl.ds(u, 1), :] = ud_ref[pl.ds(u, 1), :] + ones
                sd_ref[pl.ds(s, 1), :] = sd_ref[pl.ds(s, 1), :] + ones
            return carry

        jax.lax.fori_loop(0, c // un, step, 0)

        @pl.when(pid == nb - 1)
        def _():
            ud = ud_ref[...]
            ud_ref[...] = jnp.where(ud > 0, jax.lax.rsqrt(ud), 0.0)
            sd = sd_ref[...]
            sd_ref[...] = jnp.where(sd > 0, jax.lax.rsqrt(sd), 0.0)

    return pl.pallas_call(
        body,
        grid=(nb,),
        in_specs=[pl.BlockSpec((1, 2, c), lambda i: (i, 0, 0),
                               memory_space=pltpu.SMEM)],
        out_specs=[pl.BlockSpec((n_u, _D), lambda i: (0, 0)),
                   pl.BlockSpec((n_s, _D), lambda i: (0, 0))],
        out_shape=[jax.ShapeDtypeStruct((n_u, _D), jnp.float32),
                   jax.ShapeDtypeStruct((n_s, _D), jnp.float32)],
    )(edges.reshape(2, nb, c).swapaxes(0, 1))


def _conv_call(x, edges, w):
    m = x.shape[0]
    e = edges.shape[1]
    c = _chunk(e)
    nb = e // c

    def body(idx_ref, w_ref, x_ref, out_ref):
        @pl.when(pl.program_id(0) == 0)
        def _():
            out_ref[...] = jnp.zeros_like(out_ref)

        un = 4
        while c % un:
            un -= 1

        def step(i, carry):
            base = i * un
            for k in range(un):
                d = idx_ref[0, 0, base + k]
                s = idx_ref[0, 1, base + k]
                wv = w_ref[0, 0, base + k]
                out_ref[pl.ds(d, 1), :] = (out_ref[pl.ds(d, 1), :]
                                           + x_ref[pl.ds(s, 1), :] * wv)
            return carry

        jax.lax.fori_loop(0, c // un, step, 0)

    return pl.pallas_call(
        body,
        grid=(nb,),
        in_specs=[pl.BlockSpec((1, 2, c), lambda i: (i, 0, 0),
                               memory_space=pltpu.SMEM),
                  pl.BlockSpec((1, 1, c), lambda i: (i, 0, 0),
                               memory_space=pltpu.SMEM),
                  pl.BlockSpec((m, _D), lambda i: (0, 0))],
        out_specs=pl.BlockSpec((m, _D), lambda i: (0, 0)),
        out_shape=jax.ShapeDtypeStruct((m, _D), jnp.float32),
    )(edges.reshape(2, nb, c).swapaxes(0, 1), w.reshape(nb, 1, c), x)


def _scatter_call(edges, src_scaled, dst_scale):
    n_out = dst_scale.shape[0]
    m_src = src_scaled.shape[0]
    e = edges.shape[1]
    c = _chunk(e)
    nb = e // c
    unroll = 8
    while c % unroll:
        unroll -= 1

    def body(idx_ref, src_ref, sc_ref, out_ref):
        pid = pl.program_id(0)

        @pl.when(pid == 0)
        def _():
            out_ref[...] = jnp.zeros_like(out_ref)

        def step(i, carry):
            base = i * unroll
            for k in range(unroll):
                d = idx_ref[0, 0, base + k]
                s = idx_ref[0, 1, base + k]
                out_ref[pl.ds(d, 1), :] = (out_ref[pl.ds(d, 1), :]
                                           + src_ref[pl.ds(s, 1), :])
            return carry

        jax.lax.fori_loop(0, c // unroll, step, 0)

        @pl.when(pid == nb - 1)
        def _():
            out_ref[...] = out_ref[...] * sc_ref[...]

    return pl.pallas_call(
        body,
        grid=(nb,),
        in_specs=[pl.BlockSpec((1, 2, c), lambda i: (i, 0, 0),
                               memory_space=pltpu.SMEM),
                  pl.BlockSpec((m_src, _D), lambda i: (0, 0)),
                  pl.BlockSpec((n_out, _D), lambda i: (0, 0))],
        out_specs=pl.BlockSpec((n_out, _D), lambda i: (0, 0)),
        out_shape=jax.ShapeDtypeStruct((n_out, _D), jnp.float32),
    )(edges.reshape(2, nb, c).swapaxes(0, 1), src_scaled, dst_scale)


def _fuse_call(x, c1, c2, c3, w1, w2):
    m = x.shape[0]
    blk = 512
    nb = (m + blk - 1) // blk
    w1a = w1[:_D, 0].reshape(1, _D)
    w1b = w1[_D:, 0].reshape(1, _D)
    w2r = w2[:, 0].reshape(1, _D)

    def body(x_ref, c1_ref, c2_ref, c3_ref, a_ref, b_ref, v_ref, out_ref):
        x_b = x_ref[...]
        wa = a_ref[...]
        wb = b_ref[...]
        wv = v_ref[...]

        def score(c_b):
            t1 = jnp.sum(x_b * wa + c_b * wb, axis=1, keepdims=True)
            t2 = jnp.sum(x_b * c_b * wv, axis=1, keepdims=True)
            return jnp.exp(jnp.tanh(t1) + jnp.tanh(t2))

        cb1 = c1_ref[...]
        cb2 = c2_ref[...]
        cb3 = c3_ref[...]
        s1 = score(cb1)
        s2 = score(cb2)
        s3 = score(cb3)
        tot = s1 + s2 + s3
        out_ref[...] = x_b + (cb1 * s1 + cb2 * s2 + cb3 * s3) / tot

    row_spec = pl.BlockSpec((blk, _D), lambda i: (i, 0))
    w_spec = pl.BlockSpec((1, _D), lambda i: (0, 0))
    return pl.pallas_call(
        body,
        grid=(nb,),
        in_specs=[row_spec, row_spec, row_spec, row_spec,
                  w_spec, w_spec, w_spec],
        out_specs=row_spec,
        out_shape=jax.ShapeDtypeStruct((m, _D), jnp.float32),
    )(x, c1, c2, c3, w1a, w1b, w2r)


def _mul_call(a, b):
    m = a.shape[0]
    blk = 512
    nb = (m + blk - 1) // blk

    def body(a_ref, b_ref, out_ref):
        out_ref[...] = a_ref[...] * b_ref[...]

    spec = pl.BlockSpec((blk, _D), lambda i: (i, 0))
    return pl.pallas_call(
        body, grid=(nb,), in_specs=[spec, spec], out_specs=spec,
        out_shape=jax.ShapeDtypeStruct((m, _D), jnp.float32))(a, b)


def _addscale_call(a, b, scale):
    m = a.shape[0]
    blk = 512
    nb = (m + blk - 1) // blk

    def body(a_ref, b_ref, out_ref):
        out_ref[...] = (a_ref[...] + b_ref[...]) * scale

    spec = pl.BlockSpec((blk, _D), lambda i: (i, 0))
    return pl.pallas_call(
        body, grid=(nb,), in_specs=[spec, spec], out_specs=spec,
        out_shape=jax.ShapeDtypeStruct((m, _D), jnp.float32))(a, b)


@jax.jit
def kernel(user_spot, city_edge_index, city_edge_weight, station_edge_index,
           station_edge_weight, category_edge_index, category_edge_weight,
           user_emb, spot_emb, W1_pre, W2_pre):
    n_users = user_emb.shape[0]
    m_spots = spot_emb.shape[0]
    num_layers = 3

    c_cat = _conv_call(spot_emb, category_edge_index, category_edge_weight)
    c_cit = _conv_call(spot_emb, city_edge_index, city_edge_weight)
    c_sta = _conv_call(spot_emb, station_edge_index, station_edge_weight)
    spot_x = _fuse_call(spot_emb, c_cat, c_cit, c_sta, W1_pre, W2_pre)
    user_x = user_emb

    usd, ssd = _degree_call(user_spot, n_users, m_spots)

    us_u = user_spot
    us_s = jnp.stack([user_spot[1], user_spot[0]])

    spot_tot = spot_x
    user_tot = user_x
    for _ in range(num_layers):
        spot_s = _mul_call(spot_x, ssd)
        user_s = _mul_call(user_x, usd)
        user_x = _scatter_call(us_u, spot_s, usd)
        spot_x = _scatter_call(us_s, user_s, ssd)
        user_tot = _addscale_call(user_tot, user_x, 1.0)
        spot_tot = _addscale_call(spot_tot, spot_x, 1.0)

    inv = 1.0 / (num_layers + 1)
    spot_out = _addscale_call(spot_tot, jnp.zeros_like(spot_tot), inv)
    user_out = _addscale_call(user_tot, jnp.zeros_like(user_tot), inv)
    return spot_out, user_out
